# Initial kernel scaffold; baseline (speedup 1.0000x reference)
import functools

import jax
import jax.numpy as jnp
from jax import lax
from jax.experimental import pallas as pl
from jax.experimental.pallas import tpu as pltpu
from jax.experimental.pallas import tpu_sc as plsc

_NE = 1_600_000
_D = 32
_L = 16
_NW = 32
_C = 80
_PER_W = _NE // _NW
_N_CHUNKS = _PER_W // _C


def _tec_kernel(src_hbm, dst_hbm, off_hbm, table_hbm, out_hbm,
                sidx_v, didx_v, srows_v, drows_v, ssq_v, out_v, off_v,
                sem_s, sem_d):
    c = lax.axis_index("c")
    s = lax.axis_index("s")
    wid = s * 2 + c
    base0 = wid * _PER_W

    pltpu.sync_copy(off_hbm, off_v)
    off_relu = jnp.maximum(off_v[...], 0.0)

    def chunk_body(t, carry):
        base = base0 + t * _C
        pltpu.sync_copy(src_hbm.at[pl.ds(base, _C)], sidx_v)
        pltpu.sync_copy(dst_hbm.at[pl.ds(base, _C)], didx_v)
        cp_s = pltpu.async_copy(table_hbm.at[sidx_v], srows_v, sem_s)
        cp_d = pltpu.async_copy(table_hbm.at[didx_v], drows_v, sem_d)
        cp_s.wait()
        cp_d.wait()

        def edge_body(i, carry2):
            s0 = srows_v[i, pl.ds(0, _L)]
            s1 = srows_v[i, pl.ds(_L, _L)]
            d0 = drows_v[i, pl.ds(0, _L)]
            d1 = drows_v[i, pl.ds(_L, _L)]
            u = s0 - d0
            v = s1 - d1
            ssq_v[i] = jnp.sum(u * u + v * v, axis=0)
            return carry2

        lax.fori_loop(0, _C, edge_body, 0, unroll=4)

        def group_body(g, carry2):
            x = ssq_v[pl.ds(g * _L, _L)]
            xc = jnp.maximum(x, 1e-30)
            ib = lax.bitcast_convert_type(xc, jnp.int32)
            ib = 0x5F3759DF - lax.shift_right_logical(ib, 1)
            r = lax.bitcast_convert_type(ib, jnp.float32)
            hx = 0.5 * xc
            r = r * (1.5 - hx * r * r)
            r = r * (1.5 - hx * r * r)
            r = r * (1.5 - hx * r * r)
            dist = xc * r
            out_v[pl.ds(g * _L, _L)] = jnp.exp(off_relu - dist)
            return carry2

        lax.fori_loop(0, _C // _L, group_body, 0, unroll=5)
        pltpu.sync_copy(out_v, out_hbm.at[pl.ds(base, _C)])
        return carry

    lax.fori_loop(0, _N_CHUNKS, chunk_body, 0)


_mesh = plsc.VectorSubcoreMesh(core_axis_name="c", subcore_axis_name="s")

_poisson_sc = functools.partial(
    pl.kernel,
    mesh=_mesh,
    out_type=jax.ShapeDtypeStruct((_NE,), jnp.float32),
    scratch_types=[
        pltpu.VMEM((_C,), jnp.int32),
        pltpu.VMEM((_C,), jnp.int32),
        pltpu.VMEM((_C, _D), jnp.float32),
        pltpu.VMEM((_C, _D), jnp.float32),
        pltpu.VMEM((_C,), jnp.float32),
        pltpu.VMEM((_C,), jnp.float32),
        pltpu.VMEM((_L,), jnp.float32),
        pltpu.SemaphoreType.DMA,
        pltpu.SemaphoreType.DMA,
    ],
)(_tec_kernel)


def kernel(src, dst, offset, embedding):
    src = src.astype(jnp.int32)
    dst = dst.astype(jnp.int32)
    off16 = jnp.broadcast_to(offset.astype(jnp.float32), (_L,))
    return _poisson_sc(src, dst, off16, embedding)


# speedup vs baseline: 3.9471x; 3.9471x over previous
import functools

import jax
import jax.numpy as jnp
from jax import lax
from jax.experimental import pallas as pl
from jax.experimental.pallas import tpu as pltpu
from jax.experimental.pallas import tpu_sc as plsc

_NE = 1_600_000
_D = 32
_L = 16
_NW = 32
_C = 80
_K = 5
_B = _C * _K
_PER_W = _NE // _NW
_N_BLOCKS = _PER_W // _B


def _tec_kernel(pairs_hbm, off_hbm, table_hbm, out_hbm,
                idx_v, srows_a, srows_b, drows_a, drows_b,
                out_v, off_v,
                sem_sa, sem_sb, sem_da, sem_db):
    c = lax.axis_index("c")
    s = lax.axis_index("s")
    wid = s * 2 + c
    base0 = wid * _PER_W
    cbase0 = wid * (_PER_W // _C)

    pltpu.sync_copy(off_hbm, off_v)
    off_relu = jnp.maximum(off_v[...], 0.0)

    srows = (srows_a, srows_b)
    drows = (drows_a, drows_b)
    sem_s = (sem_sa, sem_sb)
    sem_d = (sem_da, sem_db)

    iota = lax.iota(jnp.int32, _L)

    def issue(k, buf):
        cs = pltpu.async_copy(table_hbm.at[idx_v.at[k, 0]], srows[buf], sem_s[buf])
        cd = pltpu.async_copy(table_hbm.at[idx_v.at[k, 1]], drows[buf], sem_d[buf])
        return cs, cd

    def compute(k, buf):
        sr = srows[buf]
        dr = drows[buf]

        def group_body(g, carry2):
            row = iota + g * _L
            acc0 = jnp.zeros((_L,), jnp.float32)
            acc1 = jnp.zeros((_L,), jnp.float32)
            acc2 = jnp.zeros((_L,), jnp.float32)
            acc3 = jnp.zeros((_L,), jnp.float32)
            accs = [acc0, acc1, acc2, acc3]
            for comp in range(_D):
                col = jnp.full((_L,), comp, jnp.int32)
                sv = plsc.load_gather(sr, [row, col])
                dv = plsc.load_gather(dr, [row, col])
                u = sv - dv
                accs[comp % 4] = accs[comp % 4] + u * u
            x = (accs[0] + accs[1]) + (accs[2] + accs[3])
            xc = jnp.maximum(x, 1e-30)
            ib = lax.bitcast_convert_type(xc, jnp.int32)
            ib = 0x5F3759DF - lax.shift_right_logical(ib, 1)
            r = lax.bitcast_convert_type(ib, jnp.float32)
            hx = 0.5 * xc
            r = r * (1.5 - hx * r * r)
            r = r * (1.5 - hx * r * r)
            r = r * (1.5 - hx * r * r)
            dist = xc * r
            out_v[pl.ds(k * _C + g * _L, _L)] = jnp.exp(off_relu - dist)
            return carry2

        lax.fori_loop(0, _C // _L, group_body, 0)

    def block_body(b, carry):
        base = base0 + b * _B
        pltpu.sync_copy(pairs_hbm.at[pl.ds(cbase0 + b * _K, _K)], idx_v)
        cps = issue(0, 0)
        for k in range(_K):
            nxt = None
            if k + 1 < _K:
                nxt = issue(k + 1, (k + 1) % 2)
            cps[0].wait()
            cps[1].wait()
            compute(k, k % 2)
            cps = nxt
        pltpu.sync_copy(out_v, out_hbm.at[pl.ds(base, _B)])
        return carry

    lax.fori_loop(0, _N_BLOCKS, block_body, 0)


_mesh = plsc.VectorSubcoreMesh(core_axis_name="c", subcore_axis_name="s")

_poisson_sc = functools.partial(
    pl.kernel,
    mesh=_mesh,
    compiler_params=pltpu.CompilerParams(
        needs_layout_passes=False, use_tc_tiling_on_sc=False),
    out_type=jax.ShapeDtypeStruct((_NE,), jnp.float32),
    scratch_types=[
        pltpu.VMEM((_K, 2, _C), jnp.int32),
        pltpu.VMEM((_C, _D), jnp.float32),
        pltpu.VMEM((_C, _D), jnp.float32),
        pltpu.VMEM((_C, _D), jnp.float32),
        pltpu.VMEM((_C, _D), jnp.float32),
        pltpu.VMEM((_B,), jnp.float32),
        pltpu.VMEM((_L,), jnp.float32),
        pltpu.SemaphoreType.DMA,
        pltpu.SemaphoreType.DMA,
        pltpu.SemaphoreType.DMA,
        pltpu.SemaphoreType.DMA,
    ],
)(_tec_kernel)


def kernel(src, dst, offset, embedding):
    src = src.astype(jnp.int32)
    dst = dst.astype(jnp.int32)
    pairs = jnp.stack(
        [src.reshape(-1, _C), dst.reshape(-1, _C)], axis=1)
    off16 = jnp.broadcast_to(offset.astype(jnp.float32), (_L,))
    return _poisson_sc(pairs, off16, embedding)


# speedup vs baseline: 13.9530x; 3.5350x over previous
import functools

import jax
import jax.numpy as jnp
from jax import lax
from jax.experimental import pallas as pl
from jax.experimental.pallas import tpu as pltpu
from jax.experimental.pallas import tpu_sc as plsc

_NE = 1_600_000
_NN = 100_000
_D = 32
_DW = _D // 2
_L = 16
_NW = 32
_NS = 16
_C = 80
_K = 5
_B = _C * _K
_PER_W = _NE // _NW
_N_BLOCKS = _PER_W // _B
_ROWS_PER_TILE = _NN // _NS


def _tec_kernel(pairs_hbm, off_hbm, table_hbm, out_hbm,
                tbl_sp, idx_v, srows_a, srows_b, drows_a, drows_b,
                out_v, off_v,
                sem_sa, sem_sb, sem_da, sem_db):
    c = lax.axis_index("c")
    s = lax.axis_index("s")
    wid = s * 2 + c
    base0 = wid * _PER_W
    cbase0 = wid * (_PER_W // _C)

    rbase = s * _ROWS_PER_TILE
    pltpu.sync_copy(table_hbm.at[pl.ds(rbase, _ROWS_PER_TILE)],
                    tbl_sp.at[pl.ds(rbase, _ROWS_PER_TILE)])
    plsc.subcore_barrier()

    pltpu.sync_copy(off_hbm, off_v)
    off_relu = jnp.maximum(off_v[...], 0.0)

    srows = (srows_a, srows_b)
    drows = (drows_a, drows_b)
    sem_s = (sem_sa, sem_sb)
    sem_d = (sem_da, sem_db)

    iota = lax.iota(jnp.int32, _L)

    def issue(k, buf):
        cs = pltpu.async_copy(tbl_sp.at[idx_v.at[k, 0]], srows[buf], sem_s[buf])
        cd = pltpu.async_copy(tbl_sp.at[idx_v.at[k, 1]], drows[buf], sem_d[buf])
        return cs, cd

    def compute(k, buf):
        sr = srows[buf]
        dr = drows[buf]

        def group_body(g, carry2):
            row = iota + g * _L
            acc0 = jnp.zeros((_L,), jnp.float32)
            acc1 = jnp.zeros((_L,), jnp.float32)
            acc2 = jnp.zeros((_L,), jnp.float32)
            acc3 = jnp.zeros((_L,), jnp.float32)
            accs = [acc0, acc1, acc2, acc3]
            for w in range(_DW):
                col = jnp.bitwise_and(iota + w, _DW - 1)
                sv = plsc.load_gather(sr, [row, col])
                dv = plsc.load_gather(dr, [row, col])
                sbf = plsc.bitcast(sv, jnp.bfloat16)
                dbf = plsc.bitcast(dv, jnp.bfloat16)
                ubf = sbf - dbf
                ui = plsc.bitcast(ubf, jnp.int32)
                ue = lax.bitcast_convert_type(
                    lax.shift_left(ui, 16), jnp.float32)
                uo = lax.bitcast_convert_type(
                    jnp.bitwise_and(ui, jnp.int32(-65536)), jnp.float32)
                accs[(2 * w) % 4] = accs[(2 * w) % 4] + ue * ue
                accs[(2 * w + 1) % 4] = accs[(2 * w + 1) % 4] + uo * uo
            x = (accs[0] + accs[1]) + (accs[2] + accs[3])
            xc = jnp.maximum(x, 1e-30)
            ib = lax.bitcast_convert_type(xc, jnp.int32)
            ib = 0x5F3759DF - lax.shift_right_logical(ib, 1)
            r = lax.bitcast_convert_type(ib, jnp.float32)
            hx = 0.5 * xc
            r = r * (1.5 - hx * r * r)
            r = r * (1.5 - hx * r * r)
            r = r * (1.5 - hx * r * r)
            dist = xc * r
            out_v[pl.ds(k * _C + g * _L, _L)] = jnp.exp(off_relu - dist)
            return carry2

        lax.fori_loop(0, _C // _L, group_body, 0)

    def block_body(b, carry):
        base = base0 + b * _B
        pltpu.sync_copy(pairs_hbm.at[pl.ds(cbase0 + b * _K, _K)], idx_v)
        cps = issue(0, 0)
        for k in range(_K):
            nxt = None
            if k + 1 < _K:
                nxt = issue(k + 1, (k + 1) % 2)
            cps[0].wait()
            cps[1].wait()
            compute(k, k % 2)
            cps = nxt
        pltpu.sync_copy(out_v, out_hbm.at[pl.ds(base, _B)])
        return carry

    lax.fori_loop(0, _N_BLOCKS, block_body, 0)


_mesh = plsc.VectorSubcoreMesh(core_axis_name="c", subcore_axis_name="s")

_poisson_sc = functools.partial(
    pl.kernel,
    mesh=_mesh,
    compiler_params=pltpu.CompilerParams(
        needs_layout_passes=False, use_tc_tiling_on_sc=False),
    out_type=jax.ShapeDtypeStruct((_NE,), jnp.float32),
    scratch_types=[
        pltpu.VMEM_SHARED((_NN, _DW), jnp.int32),
        pltpu.VMEM((_K, 2, _C), jnp.int32),
        pltpu.VMEM((_C, _DW), jnp.int32),
        pltpu.VMEM((_C, _DW), jnp.int32),
        pltpu.VMEM((_C, _DW), jnp.int32),
        pltpu.VMEM((_C, _DW), jnp.int32),
        pltpu.VMEM((_B,), jnp.float32),
        pltpu.VMEM((_L,), jnp.float32),
        pltpu.SemaphoreType.DMA,
        pltpu.SemaphoreType.DMA,
        pltpu.SemaphoreType.DMA,
        pltpu.SemaphoreType.DMA,
    ],
)(_tec_kernel)


def kernel(src, dst, offset, embedding):
    src = src.astype(jnp.int32)
    dst = dst.astype(jnp.int32)
    pairs = jnp.stack(
        [src.reshape(-1, _C), dst.reshape(-1, _C)], axis=1)
    table_packed = lax.bitcast_convert_type(
        embedding.astype(jnp.bfloat16).reshape(_NN, _DW, 2), jnp.int32)
    off16 = jnp.broadcast_to(offset.astype(jnp.float32), (_L,))
    return _poisson_sc(pairs, off16, table_packed)
